# baseline (speedup 1.0000x reference)
import functools

import jax
import jax.numpy as jnp
from jax import lax
from jax.experimental import pallas as pl
from jax.experimental.pallas import tpu as pltpu
from jax.experimental.pallas import tpu_sc as plsc

N = 10000
D = 128
E = 320000
EPS = 1e-5

NC = 2
NS = 16
CH = 128
KBUF = 2
NCHUNK = 80
IB = 40
EP = NC * NS * CH * NCHUNK
NPAD = 10112
DUMMY = 10048
RPI = NPAD // NS

@functools.cache
def _sc_mesh():
  return plsc.VectorSubcoreMesh(
      core_axis_name="c", subcore_axis_name="s", num_cores=NC, num_subcores=NS
  )


def _deg_body(src3, dst3, ones_s_hbm, ones_d_hbm, zeros_hbm, out_hbm, idx_s,
              idx_d, ones_v, deg_sh):
  c = lax.axis_index("c")
  s = lax.axis_index("s")
  w = c * NS + s
  base = pl.multiple_of(s * RPI, 8)
  pltpu.sync_copy(zeros_hbm, deg_sh.at[pl.ds(base, RPI)])
  pltpu.sync_copy(src3.at[w], idx_s)
  pltpu.sync_copy(dst3.at[w], idx_d)
  pltpu.sync_copy(ones_s_hbm, ones_v)
  plsc.subcore_barrier()

  @pl.loop(0, NCHUNK)
  def _(j):
    pltpu.sync_copy(ones_v, deg_sh.at[idx_s.at[j]], add=True)

  pltpu.sync_copy(ones_d_hbm, ones_v)

  @pl.loop(0, NCHUNK)
  def _(j):
    pltpu.sync_copy(ones_v, deg_sh.at[idx_d.at[j]], add=True)

  plsc.subcore_barrier()
  sl = pl.ds(base, RPI)
  pltpu.sync_copy(deg_sh.at[sl], out_hbm.at[c, sl])


@functools.cache
def _deg_call():
  return pl.kernel(
      _deg_body,
      out_type=jax.ShapeDtypeStruct((NC, NPAD, D), jnp.float32),
      mesh=_sc_mesh(),
      scratch_types=[
          pltpu.VMEM((NCHUNK, CH), jnp.int32),
          pltpu.VMEM((NCHUNK, CH), jnp.int32),
          pltpu.VMEM((CH, D), jnp.float32),
          pltpu.VMEM_SHARED((NPAD, D), jnp.float32),
      ],
  )


def _scatter_body(g_hbm, src3, dst3, zeros_hbm, out_hbm, idx_s, idx_d,
                  rows_bufs, sems, agg):
  c = lax.axis_index("c")
  s = lax.axis_index("s")
  w = c * NS + s
  base = pl.multiple_of(s * RPI, 8)
  pltpu.sync_copy(zeros_hbm, agg.at[pl.ds(base, RPI)])
  plsc.subcore_barrier()

  pltpu.sync_copy(src3.at[w], idx_s)
  pltpu.sync_copy(dst3.at[w], idx_d)

  @pl.loop(0, NCHUNK)
  def _(j):
    pltpu.async_copy(g_hbm.at[idx_s.at[j]], rows_bufs[0], sems[0]).wait()
    pltpu.sync_copy(rows_bufs[0], agg.at[idx_d.at[j]], add=True)

  plsc.subcore_barrier()
  sl = pl.ds(base, RPI)
  pltpu.sync_copy(agg.at[sl], out_hbm.at[c, sl])


@functools.cache
def _scatter_call():
  return pl.kernel(
      _scatter_body,
      out_type=jax.ShapeDtypeStruct((NC, NPAD, D), jnp.float32),
      mesh=_sc_mesh(),
      scratch_types=[
          pltpu.VMEM((NCHUNK, CH), jnp.int32),
          pltpu.VMEM((NCHUNK, CH), jnp.int32),
          [pltpu.VMEM((CH, D), jnp.float32) for _ in range(1)],
          [pltpu.SemaphoreType.DMA for _ in range(1)],
          pltpu.VMEM_SHARED((NPAD, D), jnp.float32),
      ],
  )


def _tc1_body(x_ref, deg_ref, w1_ref, out_ref):
  od = deg_ref[0, :, 0:1] + deg_ref[1, :, 0:1]
  ns = lax.rsqrt(jnp.maximum(od, 1.0))
  h = x_ref[...] * ns
  out_ref[...] = jnp.dot(h, w1_ref[...], preferred_element_type=jnp.float32)


def _tc2_body(tp_ref, deg_ref, b_ref, g_ref, be_ref, a_ref, w2_ref, out_ref):
  t = tp_ref[0] + tp_ref[1]
  idg = deg_ref[0, :, 1:2] + deg_ref[1, :, 1:2]
  nd = lax.rsqrt(jnp.maximum(idg, 1.0))
  y = t * nd + b_ref[...]
  mu = jnp.mean(y, axis=-1, keepdims=True)
  var = jnp.mean((y - mu) ** 2, axis=-1, keepdims=True)
  yn = (y - mu) * lax.rsqrt(var + EPS) * g_ref[...] + be_ref[...]
  h = jnp.where(yn > 0, yn, a_ref[0, 0] * yn)
  od = deg_ref[0, :, 0:1] + deg_ref[1, :, 0:1]
  nsrc = lax.rsqrt(jnp.maximum(od, 1.0))
  out_ref[...] = jnp.dot(h * nsrc, w2_ref[...],
                         preferred_element_type=jnp.float32)


def _tc3_body(tp_ref, deg_ref, b_ref, g_ref, be_ref, a_ref, out_ref):
  t = tp_ref[0, :N] + tp_ref[1, :N]
  idg = deg_ref[0, :, 1:2] + deg_ref[1, :, 1:2]
  nd = lax.rsqrt(jnp.maximum(idg, 1.0))[:N]
  y = t * nd + b_ref[...]
  mu = jnp.mean(y, axis=-1, keepdims=True)
  var = jnp.mean((y - mu) ** 2, axis=-1, keepdims=True)
  yn = (y - mu) * lax.rsqrt(var + EPS) * g_ref[...] + be_ref[...]
  out_ref[...] = jnp.where(yn > 0, yn, a_ref[0, 0] * yn)


_tc1 = pl.pallas_call(
    _tc1_body, out_shape=jax.ShapeDtypeStruct((NPAD, D), jnp.float32))
_tc2 = pl.pallas_call(
    _tc2_body, out_shape=jax.ShapeDtypeStruct((NPAD, D), jnp.float32))
_tc3 = pl.pallas_call(
    _tc3_body, out_shape=jax.ShapeDtypeStruct((N, D), jnp.float32))


def kernel(x, edge_index, W1, b1, g1, be1, a1, W2, b2, g2, be2, a2):
  src = edge_index[0]
  dst = edge_index[1]
  pad = jnp.full((EP - E,), DUMMY, jnp.int32)
  src3 = jnp.concatenate([src, pad]).reshape(NC * NS, NCHUNK, CH)
  dst3 = jnp.concatenate([dst, pad]).reshape(NC * NS, NCHUNK, CH)
  zerosD = jnp.zeros((RPI, D), jnp.float32)
  lane = jnp.arange(D, dtype=jnp.int32)[None, :]
  ones_s = jnp.broadcast_to((lane == 0).astype(jnp.float32), (CH, D))
  ones_d = jnp.broadcast_to((lane == 1).astype(jnp.float32), (CH, D))

  deg = _deg_call()(src3, dst3, ones_s, ones_d, zerosD)
  xpad = jnp.pad(x, ((0, NPAD - N), (0, 0)))
  ga = _tc1(xpad, deg, W1)
  t1 = _scatter_call()(ga, src3, dst3, zerosD)
  gb = _tc2(t1, deg, b1.reshape(1, D), g1.reshape(1, D), be1.reshape(1, D),
            a1.reshape(1, 1), W2)
  t2 = _scatter_call()(gb, src3, dst3, zerosD)
  return _tc3(t2, deg, b2.reshape(1, D), g2.reshape(1, D), be2.reshape(1, D),
              a2.reshape(1, 1))


# speedup vs baseline: 2.0894x; 2.0894x over previous
import functools

import jax
import jax.numpy as jnp
from jax import lax
from jax.experimental import pallas as pl
from jax.experimental.pallas import tpu as pltpu
from jax.experimental.pallas import tpu_sc as plsc

N = 10000
D = 128
E = 320000
EPS = 1e-5

NC = 2
NS = 16
CH = 128
KBUF = 2
NCHUNK = 80
IB = 40
EP = NC * NS * CH * NCHUNK
NPAD = 10112
DUMMY = 10048
RPI = NPAD // NS

@functools.cache
def _sc_mesh():
  return plsc.VectorSubcoreMesh(
      core_axis_name="c", subcore_axis_name="s", num_cores=NC, num_subcores=NS
  )


def _deg_body(src3, dst3, ones_s_hbm, ones_d_hbm, zeros_hbm, out_hbm, idx_s,
              idx_d, ones_v, deg_sh):
  c = lax.axis_index("c")
  s = lax.axis_index("s")
  w = c * NS + s
  base = pl.multiple_of(s * RPI, 8)
  pltpu.sync_copy(zeros_hbm, deg_sh.at[pl.ds(base, RPI)])
  pltpu.sync_copy(src3.at[w], idx_s)
  pltpu.sync_copy(dst3.at[w], idx_d)
  pltpu.sync_copy(ones_s_hbm, ones_v)
  plsc.subcore_barrier()

  @pl.loop(0, NCHUNK)
  def _(j):
    pltpu.sync_copy(ones_v, deg_sh.at[idx_s.at[j]], add=True)

  pltpu.sync_copy(ones_d_hbm, ones_v)

  @pl.loop(0, NCHUNK)
  def _(j):
    pltpu.sync_copy(ones_v, deg_sh.at[idx_d.at[j]], add=True)

  plsc.subcore_barrier()
  sl = pl.ds(base, RPI)
  pltpu.sync_copy(deg_sh.at[sl], out_hbm.at[c, sl])


@functools.cache
def _deg_call():
  return pl.kernel(
      _deg_body,
      out_type=jax.ShapeDtypeStruct((NC, NPAD, D), jnp.float32),
      mesh=_sc_mesh(),
      scratch_types=[
          pltpu.VMEM((NCHUNK, CH), jnp.int32),
          pltpu.VMEM((NCHUNK, CH), jnp.int32),
          pltpu.VMEM((CH, D), jnp.float32),
          pltpu.VMEM_SHARED((NPAD, D), jnp.float32),
      ],
  )


def _scatter_body(g_hbm, src3, dst3, zeros_hbm, out_hbm, idx_s, idx_d,
                  rows_bufs, sems, agg):
  c = lax.axis_index("c")
  s = lax.axis_index("s")
  w = c * NS + s
  base = pl.multiple_of(s * RPI, 8)
  pltpu.sync_copy(zeros_hbm, agg.at[pl.ds(base, RPI)])
  plsc.subcore_barrier()

  pltpu.sync_copy(src3.at[w], idx_s)
  pltpu.sync_copy(dst3.at[w], idx_d)

  @pl.loop(0, NCHUNK)
  def _(j):
    pltpu.async_copy(g_hbm.at[idx_s.at[j]], rows_bufs[0], sems[0]).wait()
    pltpu.sync_copy(rows_bufs[0], agg.at[idx_d.at[j]], add=True)

  plsc.subcore_barrier()
  sl = pl.ds(base, RPI)
  pltpu.sync_copy(agg.at[sl], out_hbm.at[c, sl])


@functools.cache
def _scatter_call():
  return pl.kernel(
      _scatter_body,
      out_type=jax.ShapeDtypeStruct((NC, NPAD, D), jnp.float32),
      mesh=_sc_mesh(),
      scratch_types=[
          pltpu.VMEM((NCHUNK, CH), jnp.int32),
          pltpu.VMEM((NCHUNK, CH), jnp.int32),
          [pltpu.VMEM((CH, D), jnp.float32) for _ in range(1)],
          [pltpu.SemaphoreType.DMA for _ in range(1)],
          pltpu.VMEM_SHARED((NPAD, D), jnp.float32),
      ],
  )


def _tc1_body(x_ref, deg_ref, w1_ref, out_ref):
  od = deg_ref[0, :, 0:1] + deg_ref[1, :, 0:1]
  ns = lax.rsqrt(jnp.maximum(od, 1.0))
  h = x_ref[...] * ns
  out_ref[...] = jnp.dot(h, w1_ref[...], preferred_element_type=jnp.float32)


def _tc2_body(tp_ref, deg_ref, b_ref, g_ref, be_ref, a_ref, w2_ref, out_ref):
  t = tp_ref[0] + tp_ref[1]
  idg = deg_ref[0, :, 1:2] + deg_ref[1, :, 1:2]
  nd = lax.rsqrt(jnp.maximum(idg, 1.0))
  y = t * nd + b_ref[...]
  mu = jnp.mean(y, axis=-1, keepdims=True)
  var = jnp.mean((y - mu) ** 2, axis=-1, keepdims=True)
  yn = (y - mu) * lax.rsqrt(var + EPS) * g_ref[...] + be_ref[...]
  h = jnp.where(yn > 0, yn, a_ref[0, 0] * yn)
  od = deg_ref[0, :, 0:1] + deg_ref[1, :, 0:1]
  nsrc = lax.rsqrt(jnp.maximum(od, 1.0))
  out_ref[...] = jnp.dot(h * nsrc, w2_ref[...],
                         preferred_element_type=jnp.float32)


def _tc3_body(tp_ref, deg_ref, b_ref, g_ref, be_ref, a_ref, out_ref):
  t = tp_ref[0, :N] + tp_ref[1, :N]
  idg = deg_ref[0, :, 1:2] + deg_ref[1, :, 1:2]
  nd = lax.rsqrt(jnp.maximum(idg, 1.0))[:N]
  y = t * nd + b_ref[...]
  mu = jnp.mean(y, axis=-1, keepdims=True)
  var = jnp.mean((y - mu) ** 2, axis=-1, keepdims=True)
  yn = (y - mu) * lax.rsqrt(var + EPS) * g_ref[...] + be_ref[...]
  out_ref[...] = jnp.where(yn > 0, yn, a_ref[0, 0] * yn)


_tc1 = pl.pallas_call(
    _tc1_body, out_shape=jax.ShapeDtypeStruct((NPAD, D), jnp.float32))
_tc2 = pl.pallas_call(
    _tc2_body, out_shape=jax.ShapeDtypeStruct((NPAD, D), jnp.float32))
_tc3 = pl.pallas_call(
    _tc3_body, out_shape=jax.ShapeDtypeStruct((N, D), jnp.float32))


def kernel(x, edge_index, W1, b1, g1, be1, a1, W2, b2, g2, be2, a2):
  src = edge_index[0]
  dst = edge_index[1]
  pad = N + (jnp.arange(EP - E, dtype=jnp.int32) % (NPAD - N))
  src3 = jnp.concatenate([src, pad]).reshape(NC * NS, NCHUNK, CH)
  dst3 = jnp.concatenate([dst, pad]).reshape(NC * NS, NCHUNK, CH)
  zerosD = jnp.zeros((RPI, D), jnp.float32)
  lane = jnp.arange(D, dtype=jnp.int32)[None, :]
  ones_s = jnp.broadcast_to((lane == 0).astype(jnp.float32), (CH, D))
  ones_d = jnp.broadcast_to((lane == 1).astype(jnp.float32), (CH, D))

  deg = _deg_call()(src3, dst3, ones_s, ones_d, zerosD)
  xpad = jnp.pad(x, ((0, NPAD - N), (0, 0)))
  ga = _tc1(xpad, deg, W1)
  t1 = _scatter_call()(ga, src3, dst3, zerosD)
  gb = _tc2(t1, deg, b1.reshape(1, D), g1.reshape(1, D), be1.reshape(1, D),
            a1.reshape(1, 1), W2)
  t2 = _scatter_call()(gb, src3, dst3, zerosD)
  return _tc3(t2, deg, b2.reshape(1, D), g2.reshape(1, D), be2.reshape(1, D),
              a2.reshape(1, 1))


# speedup vs baseline: 2.2924x; 1.0972x over previous
import functools

import jax
import jax.numpy as jnp
from jax import lax
from jax.experimental import pallas as pl
from jax.experimental.pallas import tpu as pltpu
from jax.experimental.pallas import tpu_sc as plsc

N = 10000
D = 128
E = 320000
EPS = 1e-5

NC = 2
NS = 16
CH = 128
KBUF = 2
NCHUNK = 80
IB = 40
EP = NC * NS * CH * NCHUNK
NPAD = 10112
DUMMY = 10048
RPI = NPAD // NS

@functools.cache
def _sc_mesh():
  return plsc.VectorSubcoreMesh(
      core_axis_name="c", subcore_axis_name="s", num_cores=NC, num_subcores=NS
  )


def _deg_body(src3, dst3, ones_s_hbm, ones_d_hbm, zeros_hbm, out_hbm, idx_s,
              idx_d, ones_v, deg_sh):
  c = lax.axis_index("c")
  s = lax.axis_index("s")
  w = c * NS + s
  base = pl.multiple_of(s * RPI, 8)
  pltpu.sync_copy(zeros_hbm, deg_sh.at[pl.ds(base, RPI)])
  pltpu.sync_copy(src3.at[w], idx_s)
  pltpu.sync_copy(dst3.at[w], idx_d)
  pltpu.sync_copy(ones_s_hbm, ones_v)
  plsc.subcore_barrier()

  @pl.loop(0, NCHUNK)
  def _(j):
    pltpu.sync_copy(ones_v, deg_sh.at[idx_s.at[j]], add=True)

  pltpu.sync_copy(ones_d_hbm, ones_v)

  @pl.loop(0, NCHUNK)
  def _(j):
    pltpu.sync_copy(ones_v, deg_sh.at[idx_d.at[j]], add=True)

  plsc.subcore_barrier()
  sl = pl.ds(base, RPI)
  pltpu.sync_copy(deg_sh.at[sl], out_hbm.at[c, sl])


@functools.cache
def _deg_call():
  return pl.kernel(
      _deg_body,
      out_type=jax.ShapeDtypeStruct((NC, NPAD, D), jnp.float32),
      mesh=_sc_mesh(),
      scratch_types=[
          pltpu.VMEM((NCHUNK, CH), jnp.int32),
          pltpu.VMEM((NCHUNK, CH), jnp.int32),
          pltpu.VMEM((CH, D), jnp.float32),
          pltpu.VMEM_SHARED((NPAD, D), jnp.float32),
      ],
  )


def _scatter_body(g_hbm, src3, dst3, zeros_hbm, out_hbm, idx_s, idx_d,
                  rows_bufs, sems, agg):
  c = lax.axis_index("c")
  s = lax.axis_index("s")
  w = c * NS + s
  base = pl.multiple_of(s * RPI, 8)
  pltpu.sync_copy(zeros_hbm, agg.at[pl.ds(base, RPI)])
  plsc.subcore_barrier()

  for ph in range(NCHUNK // IB):
    pltpu.sync_copy(src3.at[w, pl.ds(ph * IB, IB)], idx_s)
    pltpu.sync_copy(dst3.at[w, pl.ds(ph * IB, IB)], idx_d)

    @pl.loop(0, IB // KBUF)
    def _(h):
      j0 = h * KBUF
      descs = [
          pltpu.async_copy(g_hbm.at[idx_s.at[j0 + b]], rows_bufs[b], sems[b])
          for b in range(KBUF)
      ]
      for b in range(KBUF):
        descs[b].wait()
        pltpu.sync_copy(rows_bufs[b], agg.at[idx_d.at[j0 + b]], add=True)

  plsc.subcore_barrier()
  sl = pl.ds(base, RPI)
  pltpu.sync_copy(agg.at[sl], out_hbm.at[c, sl])


@functools.cache
def _scatter_call():
  return pl.kernel(
      _scatter_body,
      out_type=jax.ShapeDtypeStruct((NC, NPAD, D), jnp.float32),
      mesh=_sc_mesh(),
      scratch_types=[
          pltpu.VMEM((IB, CH), jnp.int32),
          pltpu.VMEM((IB, CH), jnp.int32),
          [pltpu.VMEM((CH, D), jnp.float32) for _ in range(KBUF)],
          [pltpu.SemaphoreType.DMA for _ in range(KBUF)],
          pltpu.VMEM_SHARED((NPAD, D), jnp.float32),
      ],
  )


def _tc1_body(x_ref, deg_ref, w1_ref, out_ref):
  od = deg_ref[0, :, 0:1] + deg_ref[1, :, 0:1]
  ns = lax.rsqrt(jnp.maximum(od, 1.0))
  h = x_ref[...] * ns
  out_ref[...] = jnp.dot(h, w1_ref[...], preferred_element_type=jnp.float32)


def _tc2_body(tp_ref, deg_ref, b_ref, g_ref, be_ref, a_ref, w2_ref, out_ref):
  t = tp_ref[0] + tp_ref[1]
  idg = deg_ref[0, :, 1:2] + deg_ref[1, :, 1:2]
  nd = lax.rsqrt(jnp.maximum(idg, 1.0))
  y = t * nd + b_ref[...]
  mu = jnp.mean(y, axis=-1, keepdims=True)
  var = jnp.mean((y - mu) ** 2, axis=-1, keepdims=True)
  yn = (y - mu) * lax.rsqrt(var + EPS) * g_ref[...] + be_ref[...]
  h = jnp.where(yn > 0, yn, a_ref[0, 0] * yn)
  od = deg_ref[0, :, 0:1] + deg_ref[1, :, 0:1]
  nsrc = lax.rsqrt(jnp.maximum(od, 1.0))
  out_ref[...] = jnp.dot(h * nsrc, w2_ref[...],
                         preferred_element_type=jnp.float32)


def _tc3_body(tp_ref, deg_ref, b_ref, g_ref, be_ref, a_ref, out_ref):
  t = tp_ref[0, :N] + tp_ref[1, :N]
  idg = deg_ref[0, :, 1:2] + deg_ref[1, :, 1:2]
  nd = lax.rsqrt(jnp.maximum(idg, 1.0))[:N]
  y = t * nd + b_ref[...]
  mu = jnp.mean(y, axis=-1, keepdims=True)
  var = jnp.mean((y - mu) ** 2, axis=-1, keepdims=True)
  yn = (y - mu) * lax.rsqrt(var + EPS) * g_ref[...] + be_ref[...]
  out_ref[...] = jnp.where(yn > 0, yn, a_ref[0, 0] * yn)


_tc1 = pl.pallas_call(
    _tc1_body, out_shape=jax.ShapeDtypeStruct((NPAD, D), jnp.float32))
_tc2 = pl.pallas_call(
    _tc2_body, out_shape=jax.ShapeDtypeStruct((NPAD, D), jnp.float32))
_tc3 = pl.pallas_call(
    _tc3_body, out_shape=jax.ShapeDtypeStruct((N, D), jnp.float32))


def kernel(x, edge_index, W1, b1, g1, be1, a1, W2, b2, g2, be2, a2):
  src = edge_index[0]
  dst = edge_index[1]
  pad = N + (jnp.arange(EP - E, dtype=jnp.int32) % (NPAD - N))
  src3 = jnp.concatenate([src, pad]).reshape(NC * NS, NCHUNK, CH)
  dst3 = jnp.concatenate([dst, pad]).reshape(NC * NS, NCHUNK, CH)
  zerosD = jnp.zeros((RPI, D), jnp.float32)
  lane = jnp.arange(D, dtype=jnp.int32)[None, :]
  ones_s = jnp.broadcast_to((lane == 0).astype(jnp.float32), (CH, D))
  ones_d = jnp.broadcast_to((lane == 1).astype(jnp.float32), (CH, D))

  deg = _deg_call()(src3, dst3, ones_s, ones_d, zerosD)
  xpad = jnp.pad(x, ((0, NPAD - N), (0, 0)))
  ga = _tc1(xpad, deg, W1)
  t1 = _scatter_call()(ga, src3, dst3, zerosD)
  gb = _tc2(t1, deg, b1.reshape(1, D), g1.reshape(1, D), be1.reshape(1, D),
            a1.reshape(1, 1), W2)
  t2 = _scatter_call()(gb, src3, dst3, zerosD)
  return _tc3(t2, deg, b2.reshape(1, D), g2.reshape(1, D), be2.reshape(1, D),
              a2.reshape(1, 1))


# speedup vs baseline: 2.2996x; 1.0031x over previous
import functools

import jax
import jax.numpy as jnp
from jax import lax
from jax.experimental import pallas as pl
from jax.experimental.pallas import tpu as pltpu
from jax.experimental.pallas import tpu_sc as plsc

N = 10000
D = 128
E = 320000
EPS = 1e-5

NC = 2
NS = 16
CH = 128
KBUF = 2
NCHUNK = 80
IB = 40
EP = NC * NS * CH * NCHUNK
NPAD = 10112
DUMMY = 10048
RPI = NPAD // NS

@functools.cache
def _sc_mesh():
  return plsc.VectorSubcoreMesh(
      core_axis_name="c", subcore_axis_name="s", num_cores=NC, num_subcores=NS
  )


def _deg_body(src3, dst3, ones_s_hbm, ones_d_hbm, zeros_hbm, out_hbm, idx_s,
              idx_d, ones_v, dsems, deg_sh):
  c = lax.axis_index("c")
  s = lax.axis_index("s")
  w = c * NS + s
  base = pl.multiple_of(s * RPI, 8)
  pltpu.sync_copy(zeros_hbm, deg_sh.at[pl.ds(base, RPI)])
  pltpu.sync_copy(src3.at[w], idx_s)
  pltpu.sync_copy(dst3.at[w], idx_d)
  pltpu.sync_copy(ones_s_hbm, ones_v)
  plsc.subcore_barrier()

  @pl.loop(0, NCHUNK // 2)
  def _(h):
    j0 = 2 * h
    ds = [
        pltpu.async_copy(ones_v, deg_sh.at[idx_s.at[j0 + b]], dsems[b],
                         add=True) for b in range(2)
    ]
    for b in range(2):
      ds[b].wait()

  pltpu.sync_copy(ones_d_hbm, ones_v)

  @pl.loop(0, NCHUNK // 2)
  def _(h):
    j0 = 2 * h
    ds = [
        pltpu.async_copy(ones_v, deg_sh.at[idx_d.at[j0 + b]], dsems[b],
                         add=True) for b in range(2)
    ]
    for b in range(2):
      ds[b].wait()

  plsc.subcore_barrier()
  sl = pl.ds(base, RPI)
  pltpu.sync_copy(deg_sh.at[sl], out_hbm.at[c, sl])


@functools.cache
def _deg_call():
  return pl.kernel(
      _deg_body,
      out_type=jax.ShapeDtypeStruct((NC, NPAD, D), jnp.float32),
      mesh=_sc_mesh(),
      scratch_types=[
          pltpu.VMEM((NCHUNK, CH), jnp.int32),
          pltpu.VMEM((NCHUNK, CH), jnp.int32),
          pltpu.VMEM((CH, D), jnp.float32),
          [pltpu.SemaphoreType.DMA for _ in range(2)],
          pltpu.VMEM_SHARED((NPAD, D), jnp.float32),
      ],
  )


def _scatter_body(g_hbm, src3, dst3, zeros_hbm, out_hbm, idx_s, idx_d,
                  rows_bufs, sems, agg):
  c = lax.axis_index("c")
  s = lax.axis_index("s")
  w = c * NS + s
  base = pl.multiple_of(s * RPI, 8)
  pltpu.sync_copy(zeros_hbm, agg.at[pl.ds(base, RPI)])
  plsc.subcore_barrier()

  for ph in range(NCHUNK // IB):
    pltpu.sync_copy(src3.at[w, pl.ds(ph * IB, IB)], idx_s)
    pltpu.sync_copy(dst3.at[w, pl.ds(ph * IB, IB)], idx_d)

    @pl.loop(0, IB // KBUF)
    def _(h):
      j0 = h * KBUF
      descs = [
          pltpu.async_copy(g_hbm.at[idx_s.at[j0 + b]], rows_bufs[b], sems[b])
          for b in range(KBUF)
      ]
      for b in range(KBUF):
        descs[b].wait()
        pltpu.sync_copy(rows_bufs[b], agg.at[idx_d.at[j0 + b]], add=True)

  plsc.subcore_barrier()
  sl = pl.ds(base, RPI)
  pltpu.sync_copy(agg.at[sl], out_hbm.at[c, sl])


@functools.cache
def _scatter_call():
  return pl.kernel(
      _scatter_body,
      out_type=jax.ShapeDtypeStruct((NC, NPAD, D), jnp.float32),
      mesh=_sc_mesh(),
      scratch_types=[
          pltpu.VMEM((IB, CH), jnp.int32),
          pltpu.VMEM((IB, CH), jnp.int32),
          [pltpu.VMEM((CH, D), jnp.float32) for _ in range(KBUF)],
          [pltpu.SemaphoreType.DMA for _ in range(KBUF)],
          pltpu.VMEM_SHARED((NPAD, D), jnp.float32),
      ],
  )


def _tc1_body(x_ref, deg_ref, w1_ref, out_ref):
  od = deg_ref[0, :, 0:1] + deg_ref[1, :, 0:1]
  ns = lax.rsqrt(jnp.maximum(od, 1.0))
  h = x_ref[...] * ns
  out_ref[...] = jnp.dot(h, w1_ref[...], preferred_element_type=jnp.float32)


def _tc2_body(tp_ref, deg_ref, b_ref, g_ref, be_ref, a_ref, w2_ref, out_ref):
  t = tp_ref[0] + tp_ref[1]
  idg = deg_ref[0, :, 1:2] + deg_ref[1, :, 1:2]
  nd = lax.rsqrt(jnp.maximum(idg, 1.0))
  y = t * nd + b_ref[...]
  mu = jnp.mean(y, axis=-1, keepdims=True)
  var = jnp.mean((y - mu) ** 2, axis=-1, keepdims=True)
  yn = (y - mu) * lax.rsqrt(var + EPS) * g_ref[...] + be_ref[...]
  h = jnp.where(yn > 0, yn, a_ref[0, 0] * yn)
  od = deg_ref[0, :, 0:1] + deg_ref[1, :, 0:1]
  nsrc = lax.rsqrt(jnp.maximum(od, 1.0))
  out_ref[...] = jnp.dot(h * nsrc, w2_ref[...],
                         preferred_element_type=jnp.float32)


def _tc3_body(tp_ref, deg_ref, b_ref, g_ref, be_ref, a_ref, out_ref):
  t = tp_ref[0, :N] + tp_ref[1, :N]
  idg = deg_ref[0, :, 1:2] + deg_ref[1, :, 1:2]
  nd = lax.rsqrt(jnp.maximum(idg, 1.0))[:N]
  y = t * nd + b_ref[...]
  mu = jnp.mean(y, axis=-1, keepdims=True)
  var = jnp.mean((y - mu) ** 2, axis=-1, keepdims=True)
  yn = (y - mu) * lax.rsqrt(var + EPS) * g_ref[...] + be_ref[...]
  out_ref[...] = jnp.where(yn > 0, yn, a_ref[0, 0] * yn)


_tc1 = pl.pallas_call(
    _tc1_body, out_shape=jax.ShapeDtypeStruct((NPAD, D), jnp.float32))
_tc2 = pl.pallas_call(
    _tc2_body, out_shape=jax.ShapeDtypeStruct((NPAD, D), jnp.float32))
_tc3 = pl.pallas_call(
    _tc3_body, out_shape=jax.ShapeDtypeStruct((N, D), jnp.float32))


def kernel(x, edge_index, W1, b1, g1, be1, a1, W2, b2, g2, be2, a2):
  src = edge_index[0]
  dst = edge_index[1]
  pad = N + (jnp.arange(EP - E, dtype=jnp.int32) % (NPAD - N))
  src3 = jnp.concatenate([src, pad]).reshape(NC * NS, NCHUNK, CH)
  dst3 = jnp.concatenate([dst, pad]).reshape(NC * NS, NCHUNK, CH)
  zerosD = jnp.zeros((RPI, D), jnp.float32)
  lane = jnp.arange(D, dtype=jnp.int32)[None, :]
  ones_s = jnp.broadcast_to((lane == 0).astype(jnp.float32), (CH, D))
  ones_d = jnp.broadcast_to((lane == 1).astype(jnp.float32), (CH, D))

  deg = _deg_call()(src3, dst3, ones_s, ones_d, zerosD)
  xpad = jnp.pad(x, ((0, NPAD - N), (0, 0)))
  ga = _tc1(xpad, deg, W1)
  t1 = _scatter_call()(ga, src3, dst3, zerosD)
  gb = _tc2(t1, deg, b1.reshape(1, D), g1.reshape(1, D), be1.reshape(1, D),
            a1.reshape(1, 1), W2)
  t2 = _scatter_call()(gb, src3, dst3, zerosD)
  return _tc3(t2, deg, b2.reshape(1, D), g2.reshape(1, D), be2.reshape(1, D),
              a2.reshape(1, 1))
